# Initial kernel scaffold; baseline (speedup 1.0000x reference)
import jax
import jax.numpy as jnp
from jax.experimental import pallas as pl


def kernel(x, edge_index, edge_attr, mask_node_indices, prelu_a, W_enc, W_edge, b_edge, W1, b1, gamma, beta, W2, b2):
    raise NotImplementedError("write your pallas kernel here")


# speedup vs baseline: 8.5572x; 8.5572x over previous
import functools
import jax
import jax.numpy as jnp
from jax import lax
from jax.experimental import pallas as pl
from jax.experimental.pallas import tpu as pltpu
from jax.experimental.pallas import tpu_sc as plsc

N = 10000
E = 320000
D = 128
D2 = 256
EA = 16

NC, NS = 2, 16
NW = NC * NS
EPW = E // NW
CHUNK = 128
NFULL = EPW // CHUNK
TAIL = EPW - NFULL * CHUNK
RPT = 624
RTAIL = N - NS * RPT
SROWS = 48
EROWS = 104

BLK = 1000
NBLK = N // BLK


def _enc_body(x_ref, mask_ref, wenc_ref, a_ref, o_ref):
    i = pl.program_id(0)
    xb = x_ref[...]
    a = a_ref[...]
    xb = jnp.where(xb >= 0, xb, a * xb)
    y = jnp.dot(xb, wenc_ref[...], preferred_element_type=jnp.float32)
    rows = i * BLK + lax.broadcasted_iota(jnp.int32, (BLK, 1), 0)
    hit = jnp.any(rows == mask_ref[...], axis=1, keepdims=True)
    o_ref[...] = jnp.where(hit, 0.0, y)


def _encode(x, mask_pad, W_enc, a11):
    return pl.pallas_call(
        _enc_body,
        grid=(NBLK,),
        in_specs=[
            pl.BlockSpec((BLK, D), lambda i: (i, 0)),
            pl.BlockSpec((1, 1024), lambda i: (0, 0)),
            pl.BlockSpec((D, D), lambda i: (0, 0)),
            pl.BlockSpec((1, 1), lambda i: (0, 0)),
        ],
        out_specs=pl.BlockSpec((BLK, D), lambda i: (i, 0)),
        out_shape=jax.ShapeDtypeStruct((N, D), jnp.float32),
    )(x, mask_pad, W_enc, a11)


def _sc_body(x2_hbm, src_hbm, dst_hbm, ea_hbm,
             segx_out, segea_out,
             src_v, dst_v, rows_v, ea_lin, ea_v,
             src_t, dst_t, rows_t, ea_lint, ea_t,
             stage_x, stage_e, stage_e1,
             segx_sh, segea_sh,
             s0, s1, s2, s3, s4, s5):
    c = lax.axis_index("c")
    s = lax.axis_index("s")
    wid = s * NC + c
    ebase = wid * EPW
    z16 = jnp.zeros((16,), jnp.float32)

    def zx_row(i, _):
        for j in range(D // 16):
            stage_x[i, pl.ds(j * 16, 16)] = z16
        return 0

    def ze_row(i, _):
        stage_e[i, :] = z16
        return 0

    lax.fori_loop(0, SROWS, zx_row, 0)
    lax.fori_loop(0, EROWS, ze_row, 0)

    rbase = pl.multiple_of(s * RPT, 8)
    for j in range(RPT // SROWS):
        pltpu.async_copy(stage_x,
                         segx_sh.at[pl.ds(rbase + j * SROWS, SROWS)], s0).wait()
    for j in range(RPT // EROWS):
        pltpu.async_copy(stage_e,
                         segea_sh.at[pl.ds(rbase + j * EROWS, EROWS)], s1).wait()

    @pl.when(s == NS - 1)
    def _():
        pltpu.async_copy(stage_x.at[pl.ds(0, RTAIL)],
                         segx_sh.at[pl.ds(NS * RPT, RTAIL)], s0).wait()
        pltpu.async_copy(stage_e.at[pl.ds(0, RTAIL)],
                         segea_sh.at[pl.ds(NS * RPT, RTAIL)], s1).wait()

    plsc.subcore_barrier()

    def chunk(base, sv, dv, rv, elin, ev, n):
        base = pl.multiple_of(base, 8)
        cp_s = pltpu.async_copy(src_hbm.at[pl.ds(base, n)], sv, s0)
        cp_d = pltpu.async_copy(dst_hbm.at[pl.ds(base, n)], dv, s1)
        cp_e = pltpu.async_copy(ea_hbm.at[pl.ds(base * EA, n * EA)], elin, s2)
        cp_s.wait()
        cp_g = pltpu.async_copy(x2_hbm.at[sv], rv, s3)
        cp_e.wait()
        for i in range(n):
            ev[i, :] = elin[pl.ds(i * EA, EA)]
        cp_d.wait()
        cp_g.wait()
        cx = pltpu.async_copy(rv, segx_sh.at[dv], s4, add=True)
        ce = pltpu.async_copy(ev, segea_sh.at[dv], s5, add=True)
        cx.wait()
        ce.wait()

    def body(k, _):
        chunk(ebase + k * CHUNK, src_v, dst_v, rows_v, ea_lin, ea_v, CHUNK)
        return 0

    lax.fori_loop(0, NFULL, body, 0)
    chunk(ebase + NFULL * CHUNK, src_t, dst_t, rows_t, ea_lint, ea_t, TAIL)

    plsc.subcore_barrier()
    for j in range(RPT // SROWS):
        ro = pl.multiple_of(rbase + j * SROWS, 8)
        pltpu.async_copy(segx_sh.at[pl.ds(ro, SROWS)], stage_x, s0).wait()
        pltpu.async_copy(stage_x, segx_out.at[c].at[pl.ds(ro, SROWS)],
                         s0).wait()
    def flat_row(i, _):
        stage_e1[pl.ds(i * EA, EA)] = stage_e[i, :]
        return 0

    for j in range(RPT // EROWS):
        eo = pl.multiple_of(rbase + j * EROWS, 8)
        pltpu.async_copy(segea_sh.at[pl.ds(eo, EROWS)], stage_e, s1).wait()
        lax.fori_loop(0, EROWS, flat_row, 0)
        pltpu.async_copy(stage_e1,
                         segea_out.at[c].at[pl.ds(eo * EA, EROWS * EA)],
                         s1).wait()

    @pl.when(s == NS - 1)
    def _():
        pltpu.async_copy(segx_sh.at[pl.ds(NS * RPT, RTAIL)],
                         stage_x.at[pl.ds(0, RTAIL)], s0).wait()
        pltpu.async_copy(stage_x.at[pl.ds(0, RTAIL)],
                         segx_out.at[c].at[pl.ds(NS * RPT, RTAIL)], s0).wait()
        pltpu.async_copy(segea_sh.at[pl.ds(NS * RPT, RTAIL)],
                         stage_e.at[pl.ds(0, RTAIL)], s1).wait()
        lax.fori_loop(0, RTAIL, flat_row, 0)
        pltpu.async_copy(stage_e1.at[pl.ds(0, RTAIL * EA)],
                         segea_out.at[c].at[pl.ds(NS * RPT * EA, RTAIL * EA)],
                         s1).wait()


@functools.lru_cache(maxsize=1)
def _build_sc_kernel():
    return pl.kernel(
        _sc_body,
        out_type=(jax.ShapeDtypeStruct((NC, N, D), jnp.float32),
                  jax.ShapeDtypeStruct((NC, N * EA), jnp.float32)),
        mesh=plsc.VectorSubcoreMesh(core_axis_name="c", subcore_axis_name="s"),
        compiler_params=pltpu.CompilerParams(use_tc_tiling_on_sc=False),
        scratch_types=[
            pltpu.VMEM((CHUNK,), jnp.int32),
            pltpu.VMEM((CHUNK,), jnp.int32),
            pltpu.VMEM((CHUNK, D), jnp.float32),
            pltpu.VMEM((CHUNK * EA,), jnp.float32),
            pltpu.VMEM((CHUNK, EA), jnp.float32),
            pltpu.VMEM((TAIL,), jnp.int32),
            pltpu.VMEM((TAIL,), jnp.int32),
            pltpu.VMEM((TAIL, D), jnp.float32),
            pltpu.VMEM((TAIL * EA,), jnp.float32),
            pltpu.VMEM((TAIL, EA), jnp.float32),
            pltpu.VMEM((SROWS, D), jnp.float32),
            pltpu.VMEM((EROWS, EA), jnp.float32),
            pltpu.VMEM((EROWS * EA,), jnp.float32),
            pltpu.VMEM_SHARED((N, D), jnp.float32),
            pltpu.VMEM_SHARED((N, EA), jnp.float32),
            pltpu.SemaphoreType.DMA,
            pltpu.SemaphoreType.DMA,
            pltpu.SemaphoreType.DMA,
            pltpu.SemaphoreType.DMA,
            pltpu.SemaphoreType.DMA,
            pltpu.SemaphoreType.DMA,
        ],
    )


def _sc_segment_sums(x2, src, dst, ea_flat):
    return _build_sc_kernel()(x2, src, dst, ea_flat)


def _mix_body(segx_ref, segea_ref, x2_ref, wep_ref, w1_ref, b1_ref,
              h_ref, stats_ref):
    i = pl.program_id(0)
    seg_x = segx_ref[0] + segx_ref[1] + x2_ref[...]
    col = lax.broadcasted_iota(jnp.int32, (BLK, EA), 1)
    sl = jnp.where((col == 7) | (col == 9), 1.0, 0.0)
    seg_ea = segea_ref[0] + segea_ref[1] + sl
    w1 = w1_ref[...]
    wcomb = jnp.dot(wep_ref[...], w1[D:], preferred_element_type=jnp.float32,
                 precision=lax.Precision.HIGHEST)
    h = (jnp.dot(seg_x, w1[:D], preferred_element_type=jnp.float32,
                 precision=lax.Precision.HIGHEST)
         + jnp.dot(seg_ea, wcomb, preferred_element_type=jnp.float32,
                 precision=lax.Precision.HIGHEST)
         + b1_ref[...])
    h_ref[...] = h
    st = jnp.concatenate(
        [jnp.sum(h, axis=0, keepdims=True),
         jnp.sum(h * h, axis=0, keepdims=True),
         jnp.zeros((6, D2), jnp.float32)], axis=0)

    @pl.when(i == 0)
    def _():
        stats_ref[...] = st

    @pl.when(i > 0)
    def _():
        stats_ref[...] += st


def _mix(segx_p, segea_p, x2, We_pad, W1, b1r):
    return pl.pallas_call(
        _mix_body,
        grid=(NBLK,),
        in_specs=[
            pl.BlockSpec((NC, BLK, D), lambda i: (0, i, 0)),
            pl.BlockSpec((NC, BLK, EA), lambda i: (0, i, 0)),
            pl.BlockSpec((BLK, D), lambda i: (i, 0)),
            pl.BlockSpec((EA, D), lambda i: (0, 0)),
            pl.BlockSpec((D2, D2), lambda i: (0, 0)),
            pl.BlockSpec((1, D2), lambda i: (0, 0)),
        ],
        out_specs=[
            pl.BlockSpec((BLK, D2), lambda i: (i, 0)),
            pl.BlockSpec((8, D2), lambda i: (0, 0)),
        ],
        out_shape=[
            jax.ShapeDtypeStruct((N, D2), jnp.float32),
            jax.ShapeDtypeStruct((8, D2), jnp.float32),
        ],
    )(segx_p, segea_p, x2, We_pad, W1, b1r)


def _out_body(h_ref, stats_ref, g_ref, bt_ref, w2_ref, b2_ref, o_ref):
    stats = stats_ref[...]
    mean = stats[0:1] / N
    var = stats[1:2] / N - mean * mean
    hn = (h_ref[...] - mean) * lax.rsqrt(var + 1e-5) * g_ref[...] + bt_ref[...]
    hn = jnp.maximum(hn, 0.0)
    o_ref[...] = (jnp.dot(hn, w2_ref[...], preferred_element_type=jnp.float32,
                 precision=lax.Precision.HIGHEST)
                  + b2_ref[...])


def _finish(h, stats, g, bt, W2, b2r):
    return pl.pallas_call(
        _out_body,
        grid=(NBLK,),
        in_specs=[
            pl.BlockSpec((BLK, D2), lambda i: (i, 0)),
            pl.BlockSpec((8, D2), lambda i: (0, 0)),
            pl.BlockSpec((1, D2), lambda i: (0, 0)),
            pl.BlockSpec((1, D2), lambda i: (0, 0)),
            pl.BlockSpec((D2, D), lambda i: (0, 0)),
            pl.BlockSpec((1, D), lambda i: (0, 0)),
        ],
        out_specs=pl.BlockSpec((BLK, D), lambda i: (i, 0)),
        out_shape=jax.ShapeDtypeStruct((N, D), jnp.float32),
    )(h, stats, g, bt, W2, b2r)


@jax.jit
def kernel(x, edge_index, edge_attr, mask_node_indices, prelu_a,
           W_enc, W_edge, b_edge, W1, b1, gamma, beta, W2, b2):
    src = edge_index[0].astype(jnp.int32)
    dst = edge_index[1].astype(jnp.int32)
    ea16 = jnp.zeros((E, EA), jnp.float32).at[:, :9].set(edge_attr)
    ea_flat = ea16.at[:, 9].set(1.0).reshape(-1)
    mask_pad = jnp.full((1, 1024), -1, jnp.int32)
    mask_pad = mask_pad.at[0, :1000].set(mask_node_indices.astype(jnp.int32))
    a11 = prelu_a.reshape(1, 1).astype(jnp.float32)
    We_pad = jnp.zeros((EA, D), jnp.float32).at[:9].set(W_edge).at[9].set(b_edge)
    x2 = _encode(x, mask_pad, W_enc, a11)
    segx_p, segea_f = _sc_segment_sums(x2, src, dst, ea_flat)
    segea_p = segea_f.reshape(NC, N, EA)
    h, stats = _mix(segx_p, segea_p, x2, We_pad, W1, b1.reshape(1, D2))
    return _finish(h, stats, gamma.reshape(1, D2), beta.reshape(1, D2),
                   W2, b2.reshape(1, D))


# speedup vs baseline: 9.2199x; 1.0775x over previous
import functools
import jax
import jax.numpy as jnp
from jax import lax
from jax.experimental import pallas as pl
from jax.experimental.pallas import tpu as pltpu
from jax.experimental.pallas import tpu_sc as plsc

N = 10000
E = 320000
D = 128
D2 = 256
EA = 16

NC, NS = 2, 16
NW = NC * NS
EPW = E // NW
CHUNK = 64
NFULL = EPW // CHUNK
TAIL = EPW - NFULL * CHUNK
RPT = 624
RTAIL = N - NS * RPT
SROWS = 48
EROWS = 104

BLK = 1000
NBLK = N // BLK


def _enc_body(x_ref, mask_ref, wenc_ref, a_ref, o_ref):
    i = pl.program_id(0)
    xb = x_ref[...]
    a = a_ref[...]
    xb = jnp.where(xb >= 0, xb, a * xb)
    y = jnp.dot(xb, wenc_ref[...], preferred_element_type=jnp.float32)
    rows = i * BLK + lax.broadcasted_iota(jnp.int32, (BLK, 1), 0)
    hit = jnp.any(rows == mask_ref[...], axis=1, keepdims=True)
    o_ref[...] = jnp.where(hit, 0.0, y)


def _encode(x, mask_pad, W_enc, a11):
    return pl.pallas_call(
        _enc_body,
        grid=(NBLK,),
        in_specs=[
            pl.BlockSpec((BLK, D), lambda i: (i, 0)),
            pl.BlockSpec((1, 1024), lambda i: (0, 0)),
            pl.BlockSpec((D, D), lambda i: (0, 0)),
            pl.BlockSpec((1, 1), lambda i: (0, 0)),
        ],
        out_specs=pl.BlockSpec((BLK, D), lambda i: (i, 0)),
        out_shape=jax.ShapeDtypeStruct((N, D), jnp.float32),
    )(x, mask_pad, W_enc, a11)


def _sc_body(x2_hbm, src_hbm, dst_hbm, ea_hbm,
             segx_out, segea_out,
             sv0, sv1, dv0, dv1, rv0, rv1, el0, el1, ev0, ev1,
             src_t, dst_t, rows_t, ea_lint, ea_t,
             stage_x, stage_e, stage_e1,
             segx_sh, segea_sh,
             ssl0, ssl1, sdl0, sdl1, sel0, sel1,
             sg0, sg1, sx0, sx1, sea0, sea1):
    svs = (sv0, sv1)
    dvs = (dv0, dv1)
    rvs = (rv0, rv1)
    elins = (el0, el1)
    evs = (ev0, ev1)
    ssl = (ssl0, ssl1)
    sdl = (sdl0, sdl1)
    sel = (sel0, sel1)
    sg = (sg0, sg1)
    sx = (sx0, sx1)
    sea = (sea0, sea1)
    s0, s1 = ssl0, ssl1
    c = lax.axis_index("c")
    s = lax.axis_index("s")
    wid = s * NC + c
    ebase = wid * EPW
    z16 = jnp.zeros((16,), jnp.float32)

    def zx_row(i, _):
        for j in range(D // 16):
            stage_x[i, pl.ds(j * 16, 16)] = z16
        return 0

    def ze_row(i, _):
        stage_e[i, :] = z16
        return 0

    lax.fori_loop(0, SROWS, zx_row, 0)
    lax.fori_loop(0, EROWS, ze_row, 0)

    rbase = pl.multiple_of(s * RPT, 8)
    for j in range(RPT // SROWS):
        pltpu.async_copy(stage_x,
                         segx_sh.at[pl.ds(rbase + j * SROWS, SROWS)], s0).wait()
    for j in range(RPT // EROWS):
        pltpu.async_copy(stage_e,
                         segea_sh.at[pl.ds(rbase + j * EROWS, EROWS)], s1).wait()

    @pl.when(s == NS - 1)
    def _():
        pltpu.async_copy(stage_x.at[pl.ds(0, RTAIL)],
                         segx_sh.at[pl.ds(NS * RPT, RTAIL)], s0).wait()
        pltpu.async_copy(stage_e.at[pl.ds(0, RTAIL)],
                         segea_sh.at[pl.ds(NS * RPT, RTAIL)], s1).wait()

    plsc.subcore_barrier()

    def issue_loads(base, b):
        base = pl.multiple_of(base, 8)
        pltpu.async_copy(src_hbm.at[pl.ds(base, CHUNK)], svs[b], ssl[b])
        pltpu.async_copy(dst_hbm.at[pl.ds(base, CHUNK)], dvs[b], sdl[b])
        pltpu.async_copy(ea_hbm.at[pl.ds(base * EA, CHUNK * EA)],
                         elins[b], sel[b])

    def wait_loads(base, b):
        base = pl.multiple_of(base, 8)
        pltpu.make_async_copy(src_hbm.at[pl.ds(base, CHUNK)],
                              svs[b], ssl[b]).wait()
        pltpu.make_async_copy(dst_hbm.at[pl.ds(base, CHUNK)],
                              dvs[b], sdl[b]).wait()
        pltpu.make_async_copy(ea_hbm.at[pl.ds(base * EA, CHUNK * EA)],
                              elins[b], sel[b]).wait()

    def wait_scatters(b):
        pltpu.make_async_copy(rvs[b], segx_sh.at[dvs[b]], sx[b]).wait()
        pltpu.make_async_copy(evs[b], segea_sh.at[dvs[b]], sea[b]).wait()

    def process(base, b, wait_prev, next_base):
        wait_loads(base, b)
        pltpu.async_copy(x2_hbm.at[svs[b]], rvs[b], sg[b])
        for i in range(CHUNK):
            evs[b][i, :] = elins[b][pl.ds(i * EA, EA)]
        if wait_prev:
            wait_scatters(1 - b)
        if next_base is not None:
            issue_loads(next_base, 1 - b)
        pltpu.make_async_copy(x2_hbm.at[svs[b]], rvs[b], sg[b]).wait()
        pltpu.async_copy(rvs[b], segx_sh.at[dvs[b]], sx[b], add=True)
        pltpu.async_copy(evs[b], segea_sh.at[dvs[b]], sea[b], add=True)

    issue_loads(ebase, 0)
    process(ebase, 0, False, ebase + CHUNK)

    def body(k2, _):
        kb = ebase + CHUNK + 2 * k2 * CHUNK
        process(kb, 1, True, kb + CHUNK)
        process(kb + CHUNK, 0, True, kb + 2 * CHUNK)
        return 0

    lax.fori_loop(0, (NFULL - 2) // 2, body, 0)
    process(ebase + (NFULL - 1) * CHUNK, 1, True, None)
    wait_scatters(1)

    tb = ebase + NFULL * CHUNK
    pltpu.async_copy(src_hbm.at[pl.ds(tb, TAIL)], src_t, ssl0).wait()
    pltpu.async_copy(dst_hbm.at[pl.ds(tb, TAIL)], dst_t, sdl0).wait()
    pltpu.async_copy(ea_hbm.at[pl.ds(tb * EA, TAIL * EA)], ea_lint, sel0).wait()
    pltpu.async_copy(x2_hbm.at[src_t], rows_t, sg0).wait()
    for i in range(TAIL):
        ea_t[i, :] = ea_lint[pl.ds(i * EA, EA)]
    cx = pltpu.async_copy(rows_t, segx_sh.at[dst_t], sx0, add=True)
    ce = pltpu.async_copy(ea_t, segea_sh.at[dst_t], sea0, add=True)
    cx.wait()
    ce.wait()

    plsc.subcore_barrier()
    for j in range(RPT // SROWS):
        ro = pl.multiple_of(rbase + j * SROWS, 8)
        pltpu.async_copy(segx_sh.at[pl.ds(ro, SROWS)], stage_x, s0).wait()
        pltpu.async_copy(stage_x, segx_out.at[c].at[pl.ds(ro, SROWS)],
                         s0).wait()
    def flat_row(i, _):
        stage_e1[pl.ds(i * EA, EA)] = stage_e[i, :]
        return 0

    for j in range(RPT // EROWS):
        eo = pl.multiple_of(rbase + j * EROWS, 8)
        pltpu.async_copy(segea_sh.at[pl.ds(eo, EROWS)], stage_e, s1).wait()
        lax.fori_loop(0, EROWS, flat_row, 0)
        pltpu.async_copy(stage_e1,
                         segea_out.at[c].at[pl.ds(eo * EA, EROWS * EA)],
                         s1).wait()

    @pl.when(s == NS - 1)
    def _():
        pltpu.async_copy(segx_sh.at[pl.ds(NS * RPT, RTAIL)],
                         stage_x.at[pl.ds(0, RTAIL)], s0).wait()
        pltpu.async_copy(stage_x.at[pl.ds(0, RTAIL)],
                         segx_out.at[c].at[pl.ds(NS * RPT, RTAIL)], s0).wait()
        pltpu.async_copy(segea_sh.at[pl.ds(NS * RPT, RTAIL)],
                         stage_e.at[pl.ds(0, RTAIL)], s1).wait()
        lax.fori_loop(0, RTAIL, flat_row, 0)
        pltpu.async_copy(stage_e1.at[pl.ds(0, RTAIL * EA)],
                         segea_out.at[c].at[pl.ds(NS * RPT * EA, RTAIL * EA)],
                         s1).wait()


@functools.lru_cache(maxsize=1)
def _build_sc_kernel():
    return pl.kernel(
        _sc_body,
        out_type=(jax.ShapeDtypeStruct((NC, N, D), jnp.float32),
                  jax.ShapeDtypeStruct((NC, N * EA), jnp.float32)),
        mesh=plsc.VectorSubcoreMesh(core_axis_name="c", subcore_axis_name="s"),
        compiler_params=pltpu.CompilerParams(use_tc_tiling_on_sc=False),
        scratch_types=(
            [pltpu.VMEM((CHUNK,), jnp.int32)] * 4
            + [pltpu.VMEM((CHUNK, D), jnp.float32)] * 2
            + [pltpu.VMEM((CHUNK * EA,), jnp.float32)] * 2
            + [pltpu.VMEM((CHUNK, EA), jnp.float32)] * 2
            + [
                pltpu.VMEM((TAIL,), jnp.int32),
                pltpu.VMEM((TAIL,), jnp.int32),
                pltpu.VMEM((TAIL, D), jnp.float32),
                pltpu.VMEM((TAIL * EA,), jnp.float32),
                pltpu.VMEM((TAIL, EA), jnp.float32),
                pltpu.VMEM((SROWS, D), jnp.float32),
                pltpu.VMEM((EROWS, EA), jnp.float32),
                pltpu.VMEM((EROWS * EA,), jnp.float32),
                pltpu.VMEM_SHARED((N, D), jnp.float32),
                pltpu.VMEM_SHARED((N, EA), jnp.float32),
            ]
            + [pltpu.SemaphoreType.DMA] * 12
        ),
    )


def _sc_segment_sums(x2, src, dst, ea_flat):
    return _build_sc_kernel()(x2, src, dst, ea_flat)


def _mix_body(segx_ref, segea_ref, x2_ref, wep_ref, w1_ref, b1_ref,
              g_ref, bt_ref, w2_ref, b2_ref, o_ref, h_vmem, stats_ref):
    p = pl.program_id(0)
    i = pl.program_id(1)

    @pl.when(p == 0)
    def _():
        seg_x = segx_ref[0] + segx_ref[1] + x2_ref[...]
        col = lax.broadcasted_iota(jnp.int32, (BLK, EA), 1)
        sl = jnp.where((col == 7) | (col == 9), 1.0, 0.0)
        seg_ea = segea_ref[0] + segea_ref[1] + sl
        w1 = w1_ref[...]
        wcomb = jnp.dot(wep_ref[...], w1[D:],
                        preferred_element_type=jnp.float32,
                        precision=lax.Precision.HIGHEST)
        h = (jnp.dot(seg_x, w1[:D], preferred_element_type=jnp.float32,
                     precision=lax.Precision.HIGHEST)
             + jnp.dot(seg_ea, wcomb, preferred_element_type=jnp.float32,
                       precision=lax.Precision.HIGHEST)
             + b1_ref[...])
        h_vmem[pl.ds(i * BLK, BLK), :] = h
        st = jnp.concatenate(
            [jnp.sum(h, axis=0, keepdims=True),
             jnp.sum(h * h, axis=0, keepdims=True)], axis=0)

        @pl.when(i == 0)
        def _():
            stats_ref[...] = st

        @pl.when(i > 0)
        def _():
            stats_ref[...] += st

    @pl.when(p == 1)
    def _():
        stats = stats_ref[...]
        mean = stats[0:1] / N
        var = stats[1:2] / N - mean * mean
        h = h_vmem[pl.ds(i * BLK, BLK), :]
        hn = (h - mean) * lax.rsqrt(var + 1e-5) * g_ref[...] + bt_ref[...]
        hn = jnp.maximum(hn, 0.0)
        o_ref[...] = (jnp.dot(hn, w2_ref[...],
                              preferred_element_type=jnp.float32,
                              precision=lax.Precision.HIGHEST)
                      + b2_ref[...])


def _mix(segx_p, segea_p, x2, We_pad, W1, b1r, g, bt, W2, b2r):
    return pl.pallas_call(
        _mix_body,
        grid=(2, NBLK),
        in_specs=[
            pl.BlockSpec((NC, BLK, D), lambda p, i: (0, i * (1 - p), 0)),
            pl.BlockSpec((NC, BLK, EA), lambda p, i: (0, i * (1 - p), 0)),
            pl.BlockSpec((BLK, D), lambda p, i: (i * (1 - p), 0)),
            pl.BlockSpec((EA, D), lambda p, i: (0, 0)),
            pl.BlockSpec((D2, D2), lambda p, i: (0, 0)),
            pl.BlockSpec((1, D2), lambda p, i: (0, 0)),
            pl.BlockSpec((1, D2), lambda p, i: (0, 0)),
            pl.BlockSpec((1, D2), lambda p, i: (0, 0)),
            pl.BlockSpec((D2, D), lambda p, i: (0, 0)),
            pl.BlockSpec((1, D), lambda p, i: (0, 0)),
        ],
        out_specs=pl.BlockSpec((BLK, D), lambda p, i: (i, 0)),
        out_shape=jax.ShapeDtypeStruct((N, D), jnp.float32),
        scratch_shapes=[
            pltpu.VMEM((N, D2), jnp.float32),
            pltpu.VMEM((2, D2), jnp.float32),
        ],
    )(segx_p, segea_p, x2, We_pad, W1, b1r, g, bt, W2, b2r)


@jax.jit
def kernel(x, edge_index, edge_attr, mask_node_indices, prelu_a,
           W_enc, W_edge, b_edge, W1, b1, gamma, beta, W2, b2):
    src = edge_index[0].astype(jnp.int32)
    dst = edge_index[1].astype(jnp.int32)
    ea16 = jnp.zeros((E, EA), jnp.float32).at[:, :9].set(edge_attr)
    ea_flat = ea16.at[:, 9].set(1.0).reshape(-1)
    mask_pad = jnp.full((1, 1024), -1, jnp.int32)
    mask_pad = mask_pad.at[0, :1000].set(mask_node_indices.astype(jnp.int32))
    a11 = prelu_a.reshape(1, 1).astype(jnp.float32)
    We_pad = jnp.zeros((EA, D), jnp.float32).at[:9].set(W_edge).at[9].set(b_edge)
    x2 = _encode(x, mask_pad, W_enc, a11)
    segx_p, segea_f = _sc_segment_sums(x2, src, dst, ea_flat)
    segea_p = segea_f.reshape(NC, N, EA)
    return _mix(segx_p, segea_p, x2, We_pad, W1, b1.reshape(1, D2),
                gamma.reshape(1, D2), beta.reshape(1, D2), W2,
                b2.reshape(1, D))


# speedup vs baseline: 10.4351x; 1.1318x over previous
import functools
import jax
import jax.numpy as jnp
from jax import lax
from jax.experimental import pallas as pl
from jax.experimental.pallas import tpu as pltpu
from jax.experimental.pallas import tpu_sc as plsc

N = 10000
E = 320000
D = 128
D2 = 256
EA = 16

NC, NS = 2, 16
NW = NC * NS
EPW = E // NW
CHUNK = 96
NFULL = EPW // CHUNK
TAIL = EPW - NFULL * CHUNK
NEA = 9
RPT = 624
RTAIL = N - NS * RPT
SROWS = 16
EROWS = 24

BLK = 1000
NBLK = N // BLK


def _enc_body(x_ref, mask_ref, wenc_ref, a_ref, o_ref):
    i = pl.program_id(0)
    xb = x_ref[...]
    a = a_ref[...]
    xb = jnp.where(xb >= 0, xb, a * xb)
    y = jnp.dot(xb, wenc_ref[...], preferred_element_type=jnp.float32)
    rows = i * BLK + lax.broadcasted_iota(jnp.int32, (BLK, 1), 0)
    hit = jnp.any(rows == mask_ref[...], axis=1, keepdims=True)
    o_ref[...] = jnp.where(hit, 0.0, y)


def _encode(x, mask_pad, W_enc, a11):
    return pl.pallas_call(
        _enc_body,
        grid=(NBLK,),
        in_specs=[
            pl.BlockSpec((BLK, D), lambda i: (i, 0)),
            pl.BlockSpec((1, 1024), lambda i: (0, 0)),
            pl.BlockSpec((D, D), lambda i: (0, 0)),
            pl.BlockSpec((1, 1), lambda i: (0, 0)),
        ],
        out_specs=pl.BlockSpec((BLK, D), lambda i: (i, 0)),
        out_shape=jax.ShapeDtypeStruct((N, D), jnp.float32),
    )(x, mask_pad, W_enc, a11)


def _sc_body(x2_hbm, src_hbm, dst_hbm, ea_hbm,
             segx_out, segea_out,
             sv0, sv1, dv0, dv1, rv0, rv1, el0, el1, ev0, ev1,
             src_t, dst_t,
             stage_x, stage_e, stage_e1,
             segx_sh, segea_sh,
             ssl0, ssl1, sdl0, sdl1, sel0, sel1,
             sg0, sg1, sx0, sx1, sea0, sea1):
    svs = (sv0, sv1)
    dvs = (dv0, dv1)
    rvs = (rv0, rv1)
    elins = (el0, el1)
    evs = (ev0, ev1)
    ssl = (ssl0, ssl1)
    sdl = (sdl0, sdl1)
    sel = (sel0, sel1)
    sg = (sg0, sg1)
    sx = (sx0, sx1)
    sea = (sea0, sea1)
    s0, s1 = ssl0, ssl1
    c = lax.axis_index("c")
    s = lax.axis_index("s")
    wid = s * NC + c
    ebase = wid * EPW
    z16 = jnp.zeros((16,), jnp.float32)

    def zx_row(i, _):
        for j in range(D // 16):
            stage_x[i, pl.ds(j * 16, 16)] = z16
        return 0

    def ze_row(i, _):
        stage_e[i, :] = z16
        return 0

    lax.fori_loop(0, SROWS, zx_row, 0)
    lax.fori_loop(0, EROWS, ze_row, 0)

    rbase = pl.multiple_of(s * RPT, 8)
    for j in range(RPT // SROWS):
        pltpu.async_copy(stage_x,
                         segx_sh.at[pl.ds(rbase + j * SROWS, SROWS)], s0).wait()
    for j in range(RPT // EROWS):
        pltpu.async_copy(stage_e,
                         segea_sh.at[pl.ds(rbase + j * EROWS, EROWS)], s1).wait()

    @pl.when(s == NS - 1)
    def _():
        pltpu.async_copy(stage_x.at[pl.ds(0, RTAIL)],
                         segx_sh.at[pl.ds(NS * RPT, RTAIL)], s0).wait()
        pltpu.async_copy(stage_e.at[pl.ds(0, RTAIL)],
                         segea_sh.at[pl.ds(NS * RPT, RTAIL)], s1).wait()

    plsc.subcore_barrier()

    lane = lax.iota(jnp.int32, 16)
    attr_m = lane < NEA
    deg_m = lane == NEA

    def repack(b, n):
        for i in range(n):
            v = elins[b][pl.ds(i * NEA, 16)]
            evs[b][i, :] = jnp.where(attr_m, v, jnp.where(deg_m, 1.0, 0.0))

    def issue_loads(base, b, n):
        base = pl.multiple_of(base, 8)
        pltpu.async_copy(src_hbm.at[pl.ds(base, n)], svs[b], ssl[b])
        pltpu.async_copy(dst_hbm.at[pl.ds(base, n)], dvs[b], sdl[b])
        pltpu.async_copy(ea_hbm.at[pl.ds(pl.multiple_of(base * NEA, 8),
                                         n * NEA)],
                         elins[b].at[pl.ds(0, n * NEA)], sel[b])

    def wait_loads(base, b, n):
        base = pl.multiple_of(base, 8)
        pltpu.make_async_copy(src_hbm.at[pl.ds(base, n)],
                              svs[b], ssl[b]).wait()
        pltpu.make_async_copy(dst_hbm.at[pl.ds(base, n)],
                              dvs[b], sdl[b]).wait()
        pltpu.make_async_copy(ea_hbm.at[pl.ds(base * NEA, n * NEA)],
                              elins[b].at[pl.ds(0, n * NEA)], sel[b]).wait()

    def wait_scatters(b):
        pltpu.make_async_copy(rvs[b], segx_sh.at[dvs[b]], sx[b]).wait()
        pltpu.make_async_copy(evs[b], segea_sh.at[dvs[b]], sea[b]).wait()

    def process(base, b, wait_prev, next_base):
        wait_loads(base, b, CHUNK)
        pltpu.async_copy(x2_hbm.at[svs[b]], rvs[b], sg[b])
        repack(b, CHUNK)
        if wait_prev:
            wait_scatters(1 - b)
        if next_base is not None:
            issue_loads(next_base, 1 - b, CHUNK)
        pltpu.make_async_copy(x2_hbm.at[svs[b]], rvs[b], sg[b]).wait()
        pltpu.async_copy(rvs[b], segx_sh.at[dvs[b]], sx[b], add=True)
        pltpu.async_copy(evs[b], segea_sh.at[dvs[b]], sea[b], add=True)

    issue_loads(ebase, 0, CHUNK)
    process(ebase, 0, False, ebase + CHUNK)

    def body(k2, _):
        kb = ebase + CHUNK + 2 * k2 * CHUNK
        process(kb, 1, True, kb + CHUNK)
        process(kb + CHUNK, 0, True, kb + 2 * CHUNK)
        return 0

    lax.fori_loop(0, (NFULL - 2) // 2, body, 0)
    process(ebase + (NFULL - 1) * CHUNK, 1, True, None)
    wait_scatters(1)

    tb = ebase + NFULL * CHUNK
    pltpu.async_copy(src_hbm.at[pl.ds(tb, TAIL)], src_t, ssl0).wait()
    pltpu.async_copy(dst_hbm.at[pl.ds(tb, TAIL)], dst_t, sdl0).wait()
    pltpu.async_copy(ea_hbm.at[pl.ds(pl.multiple_of(tb * NEA, 8), TAIL * NEA)],
                     el0.at[pl.ds(0, TAIL * NEA)], sel0).wait()
    pltpu.async_copy(x2_hbm.at[src_t], rv0.at[pl.ds(0, TAIL)], sg0).wait()
    repack(0, TAIL)
    cx = pltpu.async_copy(rv0.at[pl.ds(0, TAIL)], segx_sh.at[dst_t],
                          sx0, add=True)
    ce = pltpu.async_copy(ev0.at[pl.ds(0, TAIL)], segea_sh.at[dst_t],
                          sea0, add=True)
    cx.wait()
    ce.wait()

    plsc.subcore_barrier()
    for j in range(RPT // SROWS):
        ro = pl.multiple_of(rbase + j * SROWS, 8)
        pltpu.async_copy(segx_sh.at[pl.ds(ro, SROWS)], stage_x, s0).wait()
        pltpu.async_copy(stage_x, segx_out.at[c].at[pl.ds(ro, SROWS)],
                         s0).wait()
    def flat_row(i, _):
        stage_e1[pl.ds(i * EA, EA)] = stage_e[i, :]
        return 0

    for j in range(RPT // EROWS):
        eo = pl.multiple_of(rbase + j * EROWS, 8)
        pltpu.async_copy(segea_sh.at[pl.ds(eo, EROWS)], stage_e, s1).wait()
        lax.fori_loop(0, EROWS, flat_row, 0)
        pltpu.async_copy(stage_e1,
                         segea_out.at[c].at[pl.ds(eo * EA, EROWS * EA)],
                         s1).wait()

    @pl.when(s == NS - 1)
    def _():
        pltpu.async_copy(segx_sh.at[pl.ds(NS * RPT, RTAIL)],
                         stage_x.at[pl.ds(0, RTAIL)], s0).wait()
        pltpu.async_copy(stage_x.at[pl.ds(0, RTAIL)],
                         segx_out.at[c].at[pl.ds(NS * RPT, RTAIL)], s0).wait()
        pltpu.async_copy(segea_sh.at[pl.ds(NS * RPT, RTAIL)],
                         stage_e.at[pl.ds(0, RTAIL)], s1).wait()
        lax.fori_loop(0, RTAIL, flat_row, 0)
        pltpu.async_copy(stage_e1.at[pl.ds(0, RTAIL * EA)],
                         segea_out.at[c].at[pl.ds(NS * RPT * EA, RTAIL * EA)],
                         s1).wait()


@functools.lru_cache(maxsize=1)
def _build_sc_kernel():
    return pl.kernel(
        _sc_body,
        out_type=(jax.ShapeDtypeStruct((NC, N, D), jnp.float32),
                  jax.ShapeDtypeStruct((NC, N * EA), jnp.float32)),
        mesh=plsc.VectorSubcoreMesh(core_axis_name="c", subcore_axis_name="s"),
        compiler_params=pltpu.CompilerParams(use_tc_tiling_on_sc=False),
        scratch_types=(
            [pltpu.VMEM((CHUNK,), jnp.int32)] * 4
            + [pltpu.VMEM((CHUNK, D), jnp.float32)] * 2
            + [pltpu.VMEM((CHUNK * NEA + 16,), jnp.float32)] * 2
            + [pltpu.VMEM((CHUNK, EA), jnp.float32)] * 2
            + [
                pltpu.VMEM((TAIL,), jnp.int32),
                pltpu.VMEM((TAIL,), jnp.int32),
                pltpu.VMEM((SROWS, D), jnp.float32),
                pltpu.VMEM((EROWS, EA), jnp.float32),
                pltpu.VMEM((EROWS * EA,), jnp.float32),
                pltpu.VMEM_SHARED((N, D), jnp.float32),
                pltpu.VMEM_SHARED((N, EA), jnp.float32),
            ]
            + [pltpu.SemaphoreType.DMA] * 12
        ),
    )


def _sc_segment_sums(x2, src, dst, ea_flat):
    return _build_sc_kernel()(x2, src, dst, ea_flat)


def _mix_body(segx_ref, segea_ref, x2_ref, wep_ref, w1_ref, b1_ref,
              g_ref, bt_ref, w2_ref, b2_ref, o_ref, h_vmem, stats_ref):
    p = pl.program_id(0)
    i = pl.program_id(1)

    @pl.when(p == 0)
    def _():
        seg_x = segx_ref[0] + segx_ref[1] + x2_ref[...]
        col = lax.broadcasted_iota(jnp.int32, (BLK, EA), 1)
        sl = jnp.where((col == 7) | (col == 9), 1.0, 0.0)
        seg_ea = segea_ref[0] + segea_ref[1] + sl
        w1 = w1_ref[...]
        wcomb = jnp.dot(wep_ref[...], w1[D:],
                        preferred_element_type=jnp.float32,
                        precision=lax.Precision.HIGHEST)
        h = (jnp.dot(seg_x, w1[:D], preferred_element_type=jnp.float32,
                     precision=lax.Precision.HIGHEST)
             + jnp.dot(seg_ea, wcomb, preferred_element_type=jnp.float32,
                       precision=lax.Precision.HIGHEST)
             + b1_ref[...])
        h_vmem[pl.ds(i * BLK, BLK), :] = h
        st = jnp.concatenate(
            [jnp.sum(h, axis=0, keepdims=True),
             jnp.sum(h * h, axis=0, keepdims=True)], axis=0)

        @pl.when(i == 0)
        def _():
            stats_ref[...] = st

        @pl.when(i > 0)
        def _():
            stats_ref[...] += st

    @pl.when(p == 1)
    def _():
        stats = stats_ref[...]
        mean = stats[0:1] / N
        var = stats[1:2] / N - mean * mean
        h = h_vmem[pl.ds(i * BLK, BLK), :]
        hn = (h - mean) * lax.rsqrt(var + 1e-5) * g_ref[...] + bt_ref[...]
        hn = jnp.maximum(hn, 0.0)
        o_ref[...] = (jnp.dot(hn, w2_ref[...],
                              preferred_element_type=jnp.float32,
                              precision=lax.Precision.HIGHEST)
                      + b2_ref[...])


def _mix(segx_p, segea_p, x2, We_pad, W1, b1r, g, bt, W2, b2r):
    return pl.pallas_call(
        _mix_body,
        grid=(2, NBLK),
        in_specs=[
            pl.BlockSpec((NC, BLK, D), lambda p, i: (0, i * (1 - p), 0)),
            pl.BlockSpec((NC, BLK, EA), lambda p, i: (0, i * (1 - p), 0)),
            pl.BlockSpec((BLK, D), lambda p, i: (i * (1 - p), 0)),
            pl.BlockSpec((EA, D), lambda p, i: (0, 0)),
            pl.BlockSpec((D2, D2), lambda p, i: (0, 0)),
            pl.BlockSpec((1, D2), lambda p, i: (0, 0)),
            pl.BlockSpec((1, D2), lambda p, i: (0, 0)),
            pl.BlockSpec((1, D2), lambda p, i: (0, 0)),
            pl.BlockSpec((D2, D), lambda p, i: (0, 0)),
            pl.BlockSpec((1, D), lambda p, i: (0, 0)),
        ],
        out_specs=pl.BlockSpec((BLK, D), lambda p, i: (i, 0)),
        out_shape=jax.ShapeDtypeStruct((N, D), jnp.float32),
        scratch_shapes=[
            pltpu.VMEM((N, D2), jnp.float32),
            pltpu.VMEM((2, D2), jnp.float32),
        ],
    )(segx_p, segea_p, x2, We_pad, W1, b1r, g, bt, W2, b2r)


@jax.jit
def kernel(x, edge_index, edge_attr, mask_node_indices, prelu_a,
           W_enc, W_edge, b_edge, W1, b1, gamma, beta, W2, b2):
    src = edge_index[0].astype(jnp.int32)
    dst = edge_index[1].astype(jnp.int32)
    ea_flat = edge_attr.reshape(-1)
    mask_pad = jnp.full((1, 1024), -1, jnp.int32)
    mask_pad = mask_pad.at[0, :1000].set(mask_node_indices.astype(jnp.int32))
    a11 = prelu_a.reshape(1, 1).astype(jnp.float32)
    We_pad = jnp.zeros((EA, D), jnp.float32).at[:9].set(W_edge).at[9].set(b_edge)
    x2 = _encode(x, mask_pad, W_enc, a11)
    segx_p, segea_f = _sc_segment_sums(x2, src, dst, ea_flat)
    segea_p = segea_f.reshape(NC, N, EA)
    return _mix(segx_p, segea_p, x2, We_pad, W1, b1.reshape(1, D2),
                gamma.reshape(1, D2), beta.reshape(1, D2), W2,
                b2.reshape(1, D))


# speedup vs baseline: 10.5263x; 1.0087x over previous
import functools
import jax
import jax.numpy as jnp
from jax import lax
from jax.experimental import pallas as pl
from jax.experimental.pallas import tpu as pltpu
from jax.experimental.pallas import tpu_sc as plsc

N = 10000
E = 320000
D = 128
D2 = 256
EA = 16

NC, NS = 2, 16
NW = NC * NS
EPW = E // NW
CHUNK = 96
NFULL = EPW // CHUNK
TAIL = EPW - NFULL * CHUNK
NEA = 9
RPT = 624
RTAIL = N - NS * RPT
SROWS = 16
EROWS = 24

BLK = 1000
NBLK = N // BLK


def _enc_body(x_ref, mask_ref, wenc_ref, a_ref, o_ref):
    i = pl.program_id(0)
    xb = x_ref[...]
    a = a_ref[...]
    xb = jnp.where(xb >= 0, xb, a * xb)
    y = jnp.dot(xb, wenc_ref[...], preferred_element_type=jnp.float32)
    rows = i * BLK + lax.broadcasted_iota(jnp.int32, (BLK, 1), 0)
    hit = jnp.any(rows == mask_ref[...], axis=1, keepdims=True)
    o_ref[...] = jnp.where(hit, 0.0, y)


def _encode(x, mask_pad, W_enc, a11):
    return pl.pallas_call(
        _enc_body,
        grid=(NBLK,),
        in_specs=[
            pl.BlockSpec((BLK, D), lambda i: (i, 0)),
            pl.BlockSpec((1, 1024), lambda i: (0, 0)),
            pl.BlockSpec((D, D), lambda i: (0, 0)),
            pl.BlockSpec((1, 1), lambda i: (0, 0)),
        ],
        out_specs=pl.BlockSpec((BLK, D), lambda i: (i, 0)),
        out_shape=jax.ShapeDtypeStruct((N, D), jnp.float32),
    )(x, mask_pad, W_enc, a11)


def _sc_body(x2_hbm, src_hbm, dst_hbm, ea_hbm,
             segx_out, segea_out,
             sv0, sv1, dv0, dv1, rv0, rv1, el0, el1, ev0, ev1,
             src_t, dst_t,
             stage_x, stage_e, stage_ep,
             segx_sh, segea_sh,
             ssl0, ssl1, sdl0, sdl1, sel0, sel1,
             sg0, sg1, sx0, sx1, sea0, sea1):
    svs = (sv0, sv1)
    dvs = (dv0, dv1)
    rvs = (rv0, rv1)
    elins = (el0, el1)
    evs = (ev0, ev1)
    ssl = (ssl0, ssl1)
    sdl = (sdl0, sdl1)
    sel = (sel0, sel1)
    sg = (sg0, sg1)
    sx = (sx0, sx1)
    sea = (sea0, sea1)
    s0, s1 = ssl0, ssl1
    c = lax.axis_index("c")
    s = lax.axis_index("s")
    wid = s * NC + c
    ebase = wid * EPW
    z16 = jnp.zeros((16,), jnp.float32)

    def zx_row(i, _):
        for j in range(D // 16):
            stage_x[i, pl.ds(j * 16, 16)] = z16
        return 0

    def ze_row(i, _):
        stage_e[i, :] = z16
        for j in range(D // 16):
            stage_ep[i, pl.ds(j * 16, 16)] = z16
        return 0

    lax.fori_loop(0, SROWS, zx_row, 0)
    lax.fori_loop(0, EROWS, ze_row, 0)

    rbase = pl.multiple_of(s * RPT, 8)
    for j in range(RPT // SROWS):
        pltpu.async_copy(stage_x,
                         segx_sh.at[pl.ds(rbase + j * SROWS, SROWS)], s0).wait()
    for j in range(RPT // EROWS):
        pltpu.async_copy(stage_e,
                         segea_sh.at[pl.ds(rbase + j * EROWS, EROWS)], s1).wait()

    @pl.when(s == NS - 1)
    def _():
        pltpu.async_copy(stage_x.at[pl.ds(0, RTAIL)],
                         segx_sh.at[pl.ds(NS * RPT, RTAIL)], s0).wait()
        pltpu.async_copy(stage_e.at[pl.ds(0, RTAIL)],
                         segea_sh.at[pl.ds(NS * RPT, RTAIL)], s1).wait()

    plsc.subcore_barrier()

    lane = lax.iota(jnp.int32, 16)
    attr_m = lane < NEA
    deg_m = lane == NEA

    def repack(b, n):
        for i in range(n):
            v = elins[b][pl.ds(i * NEA, 16)]
            evs[b][i, :] = jnp.where(attr_m, v, jnp.where(deg_m, 1.0, 0.0))

    def issue_loads(base, b, n):
        base = pl.multiple_of(base, 8)
        pltpu.async_copy(src_hbm.at[pl.ds(base, n)], svs[b], ssl[b])
        pltpu.async_copy(dst_hbm.at[pl.ds(base, n)], dvs[b], sdl[b])
        pltpu.async_copy(ea_hbm.at[pl.ds(pl.multiple_of(base * NEA, 8),
                                         n * NEA)],
                         elins[b].at[pl.ds(0, n * NEA)], sel[b])

    def wait_loads(base, b, n):
        base = pl.multiple_of(base, 8)
        pltpu.make_async_copy(src_hbm.at[pl.ds(base, n)],
                              svs[b], ssl[b]).wait()
        pltpu.make_async_copy(dst_hbm.at[pl.ds(base, n)],
                              dvs[b], sdl[b]).wait()
        pltpu.make_async_copy(ea_hbm.at[pl.ds(base * NEA, n * NEA)],
                              elins[b].at[pl.ds(0, n * NEA)], sel[b]).wait()

    def wait_scatters(b):
        pltpu.make_async_copy(rvs[b], segx_sh.at[dvs[b]], sx[b]).wait()
        pltpu.make_async_copy(evs[b], segea_sh.at[dvs[b]], sea[b]).wait()

    def process(base, b, wait_prev, next_base):
        wait_loads(base, b, CHUNK)
        pltpu.async_copy(x2_hbm.at[svs[b]], rvs[b], sg[b])
        repack(b, CHUNK)
        if wait_prev:
            wait_scatters(1 - b)
        if next_base is not None:
            issue_loads(next_base, 1 - b, CHUNK)
        pltpu.make_async_copy(x2_hbm.at[svs[b]], rvs[b], sg[b]).wait()
        pltpu.async_copy(rvs[b], segx_sh.at[dvs[b]], sx[b], add=True)
        pltpu.async_copy(evs[b], segea_sh.at[dvs[b]], sea[b], add=True)

    issue_loads(ebase, 0, CHUNK)
    process(ebase, 0, False, ebase + CHUNK)

    def body(k2, _):
        kb = ebase + CHUNK + 2 * k2 * CHUNK
        process(kb, 1, True, kb + CHUNK)
        process(kb + CHUNK, 0, True, kb + 2 * CHUNK)
        return 0

    lax.fori_loop(0, (NFULL - 2) // 2, body, 0)
    process(ebase + (NFULL - 1) * CHUNK, 1, True, None)
    wait_scatters(1)

    tb = ebase + NFULL * CHUNK
    pltpu.async_copy(src_hbm.at[pl.ds(tb, TAIL)], src_t, ssl0).wait()
    pltpu.async_copy(dst_hbm.at[pl.ds(tb, TAIL)], dst_t, sdl0).wait()
    pltpu.async_copy(ea_hbm.at[pl.ds(pl.multiple_of(tb * NEA, 8), TAIL * NEA)],
                     el0.at[pl.ds(0, TAIL * NEA)], sel0).wait()
    pltpu.async_copy(x2_hbm.at[src_t], rv0.at[pl.ds(0, TAIL)], sg0).wait()
    repack(0, TAIL)
    cx = pltpu.async_copy(rv0.at[pl.ds(0, TAIL)], segx_sh.at[dst_t],
                          sx0, add=True)
    ce = pltpu.async_copy(ev0.at[pl.ds(0, TAIL)], segea_sh.at[dst_t],
                          sea0, add=True)
    cx.wait()
    ce.wait()

    plsc.subcore_barrier()
    for j in range(RPT // SROWS):
        ro = pl.multiple_of(rbase + j * SROWS, 8)
        pltpu.async_copy(segx_sh.at[pl.ds(ro, SROWS)], stage_x, s0).wait()
        pltpu.async_copy(stage_x, segx_out.at[c].at[pl.ds(ro, SROWS)],
                         s0).wait()
    def pad_row(i, _):
        stage_ep[i, pl.ds(0, EA)] = stage_e[i, :]
        return 0

    for j in range(RPT // EROWS):
        eo = pl.multiple_of(rbase + j * EROWS, 8)
        pltpu.async_copy(segea_sh.at[pl.ds(eo, EROWS)], stage_e, s1).wait()
        lax.fori_loop(0, EROWS, pad_row, 0)
        pltpu.async_copy(stage_ep,
                         segea_out.at[c].at[pl.ds(eo, EROWS)], s1).wait()

    @pl.when(s == NS - 1)
    def _():
        pltpu.async_copy(segx_sh.at[pl.ds(NS * RPT, RTAIL)],
                         stage_x.at[pl.ds(0, RTAIL)], s0).wait()
        pltpu.async_copy(stage_x.at[pl.ds(0, RTAIL)],
                         segx_out.at[c].at[pl.ds(NS * RPT, RTAIL)], s0).wait()
        pltpu.async_copy(segea_sh.at[pl.ds(NS * RPT, RTAIL)],
                         stage_e.at[pl.ds(0, RTAIL)], s1).wait()
        lax.fori_loop(0, RTAIL, pad_row, 0)
        pltpu.async_copy(stage_ep.at[pl.ds(0, RTAIL)],
                         segea_out.at[c].at[pl.ds(NS * RPT, RTAIL)],
                         s1).wait()


@functools.lru_cache(maxsize=1)
def _build_sc_kernel():
    return pl.kernel(
        _sc_body,
        out_type=(jax.ShapeDtypeStruct((NC, N, D), jnp.float32),
                  jax.ShapeDtypeStruct((NC, N, D), jnp.float32)),
        mesh=plsc.VectorSubcoreMesh(core_axis_name="c", subcore_axis_name="s"),
        compiler_params=pltpu.CompilerParams(use_tc_tiling_on_sc=False),
        scratch_types=(
            [pltpu.VMEM((CHUNK,), jnp.int32)] * 4
            + [pltpu.VMEM((CHUNK, D), jnp.float32)] * 2
            + [pltpu.VMEM((CHUNK * NEA + 16,), jnp.float32)] * 2
            + [pltpu.VMEM((CHUNK, EA), jnp.float32)] * 2
            + [
                pltpu.VMEM((TAIL,), jnp.int32),
                pltpu.VMEM((TAIL,), jnp.int32),
                pltpu.VMEM((SROWS, D), jnp.float32),
                pltpu.VMEM((EROWS, EA), jnp.float32),
                pltpu.VMEM((EROWS, D), jnp.float32),
                pltpu.VMEM_SHARED((N, D), jnp.float32),
                pltpu.VMEM_SHARED((N, EA), jnp.float32),
            ]
            + [pltpu.SemaphoreType.DMA] * 12
        ),
    )


def _sc_segment_sums(x2, src, dst, ea_flat):
    return _build_sc_kernel()(x2, src, dst, ea_flat)


def _mix_body(segx_ref, segea_ref, x2_ref, wep_ref, w1_ref, b1_ref,
              g_ref, bt_ref, w2_ref, b2_ref, o_ref, h_vmem, stats_ref):
    p = pl.program_id(0)
    i = pl.program_id(1)

    @pl.when(p == 0)
    def _():
        seg_x = segx_ref[0] + segx_ref[1] + x2_ref[...]
        col = lax.broadcasted_iota(jnp.int32, (BLK, D), 1)
        sl = jnp.where((col == 7) | (col == 9), 1.0, 0.0)
        seg_ea = segea_ref[0] + segea_ref[1] + sl
        w1 = w1_ref[...]
        wcomb = jnp.dot(wep_ref[...], w1[D:],
                        preferred_element_type=jnp.float32,
                        precision=lax.Precision.HIGHEST)
        h = (jnp.dot(seg_x, w1[:D], preferred_element_type=jnp.float32,
                     precision=lax.Precision.HIGHEST)
             + jnp.dot(seg_ea, wcomb, preferred_element_type=jnp.float32,
                       precision=lax.Precision.HIGHEST)
             + b1_ref[...])
        h_vmem[pl.ds(i * BLK, BLK), :] = h
        st = jnp.concatenate(
            [jnp.sum(h, axis=0, keepdims=True),
             jnp.sum(h * h, axis=0, keepdims=True)], axis=0)

        @pl.when(i == 0)
        def _():
            stats_ref[...] = st

        @pl.when(i > 0)
        def _():
            stats_ref[...] += st

    @pl.when(p == 1)
    def _():
        stats = stats_ref[...]
        mean = stats[0:1] / N
        var = stats[1:2] / N - mean * mean
        h = h_vmem[pl.ds(i * BLK, BLK), :]
        hn = (h - mean) * lax.rsqrt(var + 1e-5) * g_ref[...] + bt_ref[...]
        hn = jnp.maximum(hn, 0.0)
        o_ref[...] = (jnp.dot(hn, w2_ref[...],
                              preferred_element_type=jnp.float32,
                              precision=lax.Precision.HIGHEST)
                      + b2_ref[...])


def _mix(segx_p, segea_p, x2, We_pad, W1, b1r, g, bt, W2, b2r):
    return pl.pallas_call(
        _mix_body,
        grid=(2, NBLK),
        in_specs=[
            pl.BlockSpec((NC, BLK, D), lambda p, i: (0, i * (1 - p), 0)),
            pl.BlockSpec((NC, BLK, D), lambda p, i: (0, i * (1 - p), 0)),
            pl.BlockSpec((BLK, D), lambda p, i: (i * (1 - p), 0)),
            pl.BlockSpec((D, D), lambda p, i: (0, 0)),
            pl.BlockSpec((D2, D2), lambda p, i: (0, 0)),
            pl.BlockSpec((1, D2), lambda p, i: (0, 0)),
            pl.BlockSpec((1, D2), lambda p, i: (0, 0)),
            pl.BlockSpec((1, D2), lambda p, i: (0, 0)),
            pl.BlockSpec((D2, D), lambda p, i: (0, 0)),
            pl.BlockSpec((1, D), lambda p, i: (0, 0)),
        ],
        out_specs=pl.BlockSpec((BLK, D), lambda p, i: (i, 0)),
        out_shape=jax.ShapeDtypeStruct((N, D), jnp.float32),
        scratch_shapes=[
            pltpu.VMEM((N, D2), jnp.float32),
            pltpu.VMEM((2, D2), jnp.float32),
        ],
    )(segx_p, segea_p, x2, We_pad, W1, b1r, g, bt, W2, b2r)


@jax.jit
def kernel(x, edge_index, edge_attr, mask_node_indices, prelu_a,
           W_enc, W_edge, b_edge, W1, b1, gamma, beta, W2, b2):
    src = edge_index[0].astype(jnp.int32)
    dst = edge_index[1].astype(jnp.int32)
    ea_flat = edge_attr.reshape(-1)
    mask_pad = jnp.full((1, 1024), -1, jnp.int32)
    mask_pad = mask_pad.at[0, :1000].set(mask_node_indices.astype(jnp.int32))
    a11 = prelu_a.reshape(1, 1).astype(jnp.float32)
    We_pad = jnp.zeros((D, D), jnp.float32).at[:9].set(W_edge).at[9].set(b_edge)
    x2 = _encode(x, mask_pad, W_enc, a11)
    segx_p, segea_p = _sc_segment_sums(x2, src, dst, ea_flat)
    return _mix(segx_p, segea_p, x2, We_pad, W1, b1.reshape(1, D2),
                gamma.reshape(1, D2), beta.reshape(1, D2), W2,
                b2.reshape(1, D))
